# Initial kernel scaffold; baseline (speedup 1.0000x reference)
import jax
import jax.numpy as jnp
from jax.experimental import pallas as pl


def kernel(x, edge_index, edge_attrs):
    raise NotImplementedError("write your pallas kernel here")


# speedup vs baseline: 6.8548x; 6.8548x over previous
import jax
import jax.numpy as jnp
from jax import lax
from jax.experimental import pallas as pl
from jax.experimental.pallas import tpu as pltpu
from jax.experimental.pallas import tpu_sc as plsc

N = 10000
E = 320000
D = 128
DH = D // 2
NC = 2
NS = 16
L = 16
NG = 5
NPAD = 10240
TRASH = NPAD - 1
RPT = NPAD // NS
ET = E // NS
C = 80
SC_CH = 10
NSUPER = ET // (C * SC_CH)
BLK = 80
CP = 96
NBLK = RPT // BLK


def _body(xa, xb, from_e, to_e, attr_e, out,
          fr10, to10, at10, fr1, trx, tr0, tr1, tr2, tr3, tr4,
          dbuf, ones_r, w_v, rows_v, osp, deg16):
  tr = (tr0, tr1, tr2, tr3, tr4)
  c = lax.axis_index("c")
  s = lax.axis_index("s")
  base = s * RPT
  zeros16 = jnp.zeros((L,), jnp.float32)
  ones16 = jnp.ones((L,), jnp.float32)
  iota16 = lax.iota(jnp.int32, L)

  def _occurrence(tg):
    trx[:] = tg
    occ = jnp.zeros((L,), jnp.int32)
    for sh in range(1, L):
      rot = plsc.load_gather(trx, [(iota16 - sh) & (L - 1)])
      m = (tg == rot) & (iota16 >= sh)
      occ = occ + jnp.where(m, 1, 0)
    return occ

  def _prep_groups(idx_ref, jj):
    groups = []
    for g in range(NG):
      tg = idx_ref[jj, pl.ds(g * L, L)]
      occ = _occurrence(tg)
      tr[g][:] = jnp.where(occ == 0, tg, TRASH)
      groups.append((tg, occ))
    return groups

  def _fill(e, _):
    dbuf[e, :] = zeros16
    for f in range(DH // L):
      rows_v[e, pl.ds(f * L, L)] = zeros16
    return 0
  lax.fori_loop(0, CP, _fill, 0)
  for e in range(L):
    ones_r[e, :] = ones16

  def _zz(b, _):
    pltpu.sync_copy(rows_v.at[pl.ds(0, BLK)], osp.at[pl.ds(base + b * BLK, BLK)])
    pltpu.sync_copy(dbuf.at[pl.ds(0, BLK)], deg16.at[pl.ds(base + b * BLK, BLK)])
    return 0
  lax.fori_loop(0, NBLK, _zz, 0)
  plsc.subcore_barrier()

  def _deg_super(J, _):
    pltpu.sync_copy(to_e.at[s, J], to10)

    def _deg_chunk(jj, _):
      groups = _prep_groups(to10, jj)
      for g in range(NG):
        pltpu.sync_copy(ones_r.at[pl.ds(0, L)], deg16.at[tr[g]], add=True)
      for g, (tg, occ) in enumerate(groups):
        maxocc = jnp.max(occ)

        def _round(r, _):
          trx[:] = jnp.where(occ == r, tg, TRASH)
          pl.delay(100)
          pltpu.sync_copy(ones_r.at[pl.ds(0, L)], deg16.at[trx], add=True)
          return 0
        lax.fori_loop(1, maxocc + 1, _round, 0)
      return 0
    lax.fori_loop(0, SC_CH, _deg_chunk, 0)
    return 0
  lax.fori_loop(0, NSUPER, _deg_super, 0)
  plsc.subcore_barrier()

  def _dis(b, _):
    r0 = base + b * BLK
    pltpu.sync_copy(deg16.at[pl.ds(r0, BLK)], dbuf.at[pl.ds(0, BLK)])
    for e in range(BLK):
      d = dbuf[e, :]
      bits = plsc.bitcast(d, jnp.int32)
      y = plsc.bitcast(jnp.int32(0x5F3759DF) - (bits >> 1), jnp.float32)
      for _ in range(3):
        y = y * (1.5 - 0.5 * d * y * y)
      dbuf[e, :] = jnp.where(d > 0.5, y, 0.0)
    pltpu.sync_copy(dbuf.at[pl.ds(0, BLK)], deg16.at[pl.ds(r0, BLK)])
    return 0
  lax.fori_loop(0, NBLK, _dis, 0)
  plsc.subcore_barrier()

  def _main_super(J, _):
    pltpu.sync_copy(from_e.at[s, J], fr10)
    pltpu.sync_copy(to_e.at[s, J], to10)
    pltpu.sync_copy(attr_e.at[s, J], at10)

    def _chunk(jj, _):
      for g in range(CP // L):
        sh_idx = jnp.clip(g * L + iota16 - 1, 0, C - 1)
        fr1[pl.ds(g * L, L)] = plsc.load_gather(
            fr10, [jnp.full((L,), jj, jnp.int32), sh_idx])
      groups = _prep_groups(to10, jj)

      @pl.when(c == 0)
      def _():
        pltpu.sync_copy(xa.at[fr1], rows_v)

      @pl.when(c == 1)
      def _():
        pltpu.sync_copy(xb.at[fr1], rows_v)

      pltpu.sync_copy(deg16.at[fr1], dbuf)

      for g in range(NG):
        gs = pl.ds(g * L, L)
        a = at10[jj, gs]
        df = plsc.load_gather(
            dbuf, [1 + g * L + iota16, jnp.zeros((L,), jnp.int32)])
        w_v[pl.ds(L + g * L, L)] = df * (1.0 + jnp.exp(-a))

      for e in range(C):
        sp = plsc.load_gather(w_v, [jnp.full((L,), L + e, jnp.int32)])
        for f in range(DH // L):
          sl = pl.ds(f * L, L)
          rows_v[1 + e, sl] = rows_v[1 + e, sl] * sp

      for g in range(NG):
        pltpu.sync_copy(
            rows_v.at[pl.ds(1 + g * L, L)], osp.at[tr[g]], add=True)
      for g, (tg, occ) in enumerate(groups):
        maxocc = jnp.max(occ)

        def _round(r, _):
          trx[:] = jnp.where(occ == r, tg, TRASH)
          pl.delay(100)
          pltpu.sync_copy(
              rows_v.at[pl.ds(1 + g * L, L)], osp.at[trx], add=True)
          return 0
        lax.fori_loop(1, maxocc + 1, _round, 0)
      return 0
    lax.fori_loop(0, SC_CH, _chunk, 0)
    return 0
  lax.fori_loop(0, NSUPER, _main_super, 0)
  plsc.subcore_barrier()

  def _po(b, _):
    r0 = base + b * BLK
    pltpu.sync_copy(osp.at[pl.ds(r0, BLK)], rows_v.at[pl.ds(0, BLK)])
    pltpu.sync_copy(deg16.at[pl.ds(r0, BLK)], dbuf.at[pl.ds(0, BLK)])
    for e in range(BLK):
      sp = dbuf[e, :]
      for f in range(DH // L):
        sl = pl.ds(f * L, L)
        rows_v[e, sl] = rows_v[e, sl] * sp
    pltpu.sync_copy(rows_v.at[pl.ds(0, BLK)], out.at[c, pl.ds(r0, BLK)])
    return 0
  lax.fori_loop(0, NBLK, _po, 0)


_gnn = pl.kernel(
    _body,
    out_type=jax.ShapeDtypeStruct((NC, NPAD, DH), jnp.float32),
    mesh=plsc.VectorSubcoreMesh(core_axis_name="c", subcore_axis_name="s"),
    compiler_params=pltpu.CompilerParams(
        needs_layout_passes=False, use_tc_tiling_on_sc=False),
    scratch_types=[
        pltpu.VMEM((SC_CH, C), jnp.int32),
        pltpu.VMEM((SC_CH, C), jnp.int32),
        pltpu.VMEM((SC_CH, C), jnp.float32),
        pltpu.VMEM((CP,), jnp.int32),
        pltpu.VMEM((L,), jnp.int32),
        pltpu.VMEM((L,), jnp.int32),
        pltpu.VMEM((L,), jnp.int32),
        pltpu.VMEM((L,), jnp.int32),
        pltpu.VMEM((L,), jnp.int32),
        pltpu.VMEM((L,), jnp.int32),
        pltpu.VMEM((CP, L), jnp.float32),
        pltpu.VMEM((L, L), jnp.float32),
        pltpu.VMEM((C + L,), jnp.float32),
        pltpu.VMEM((CP, DH), jnp.float32),
        pltpu.VMEM_SHARED((NPAD, DH), jnp.float32),
        pltpu.VMEM_SHARED((NPAD, L), jnp.float32),
    ],
)


def kernel(x, edge_index, edge_attrs):
  ei = edge_index.astype(jnp.int32)
  from_e = ei[0].reshape(NS, NSUPER, SC_CH, C)
  to_e = ei[1].reshape(NS, NSUPER, SC_CH, C)
  attr_e = edge_attrs.astype(jnp.float32).reshape(NS, NSUPER, SC_CH, C)
  xa = jnp.pad(x[:, :DH], ((0, NPAD - N), (0, 0)))
  xb = jnp.pad(x[:, DH:], ((0, NPAD - N), (0, 0)))
  out2 = _gnn(xa, xb, from_e, to_e, attr_e)
  return jnp.concatenate([out2[0, :N], out2[1, :N]], axis=1)
